# baseline (speedup 1.0000x reference)
import jax
import jax.numpy as jnp
from jax import lax
from jax.experimental import pallas as pl
from jax.experimental.pallas import tpu as pltpu
from jax.experimental.pallas import tpu_sc as plsc

_DIM = 64
_LANES = 128
_NC = 2
_NS = 16
_NW = _NC * _NS


def _gather_body(tbl_hbm, idx_hbm, out_hbm,
                 idx_slab, glist, par, gbuf, obuf,
                 gsem0, gsem1, wsem0, wsem1):
    L = out_hbm.shape[0]
    BT = out_hbm.shape[2]
    bcols = BT // _NW
    mt_per_w = bcols // _LANES
    n_steps = L * mt_per_w
    slab = bcols * L

    wid = lax.axis_index("s") * _NC + lax.axis_index("c")
    b0 = wid * bcols

    gsems = (gsem0, gsem1)
    wsems = (wsem0, wsem1)

    pltpu.sync_copy(idx_hbm.at[pl.ds(wid * slab, slab)], idx_slab)

    iota = lax.iota(jnp.int32, 16)

    def build_lists(t, b):
        l = t >> 2
        mm = t & (mt_per_w - 1)
        for g in range(8):
            offs = (mm * _LANES + g * 16 + iota) * L + l
            v = plsc.load_gather(idx_slab, [offs])
            row = lax.shift_right_logical(v, 1)
            parv = lax.shift_left(lax.bitwise_and(v, 1), 6)
            glist.at[b][pl.ds(g * 16, 16)] = row
            par.at[b][pl.ds(g * 16, 16)] = parv

    def start_gather(b):
        return pltpu.async_copy(tbl_hbm.at[glist.at[b]], gbuf.at[b], gsems[b])

    def transpose(b):
        par_ref = par.at[b]
        gb = gbuf.at[b]
        ob = obuf.at[b]
        rowvs = [g * 16 + iota for g in range(8)]
        parvs = [plsc.load_gather(par_ref, [rowvs[g]]) for g in range(8)]

        @plsc.parallel_loop(0, _DIM, unroll=8)
        def _(c):
            for g in range(8):
                val = plsc.load_gather(gb, [rowvs[g], parvs[g] + c])
                ob[c, pl.ds(g * 16, 16)] = val

    def start_write(t, b):
        l = t >> 2
        mm = t & (mt_per_w - 1)
        return pltpu.async_copy(
            obuf.at[b],
            out_hbm.at[l, :, pl.ds(b0 + mm * _LANES, _LANES)],
            wsems[b],
        )

    def wait_write(b):
        pltpu.make_async_copy(
            obuf.at[b], out_hbm.at[0, :, pl.ds(b0, _LANES)], wsems[b]
        ).wait()

    @pl.loop(0, n_steps // 2)
    def _(s):
        descs = []
        for b in range(2):
            t = 2 * s + b

            @pl.when(s > 0)
            def _():
                wait_write(b)

            build_lists(t, b)
            descs.append(start_gather(b))
        for b in range(2):
            t = 2 * s + b
            descs[b].wait()
            transpose(b)
            start_write(t, b)

    for b in range(2):
        wait_write(b)


def _relayout_body(tblt_hbm, tail_hbm, tbl2_hbm, tin, tout, tailv,
                   rsem0, rsem1, wsem0, wsem1):
    D, V = tblt_hbm.shape
    full_cols = V // _LANES
    pairs = full_cols // 2
    base_p = pairs // _NW
    remp = pairs - base_p * _NW

    wid = lax.axis_index("s") * _NC + lax.axis_index("c")
    start = 2 * (wid * base_p + lax.min(wid, remp))
    count = 2 * (base_p + jnp.where(wid < remp, 1, 0))

    rsems = (rsem0, rsem1)
    wsems = (wsem0, wsem1)
    iota = lax.iota(jnp.int32, 16)

    @pl.when(wid == _NW - 1)
    def _():
        pltpu.sync_copy(tail_hbm, tailv)
        pltpu.sync_copy(
            tailv, tbl2_hbm.at[pl.ds(full_cols * (_LANES // 2),
                                     tail_hbm.shape[0]), :])

    def start_read(g, b):
        return pltpu.async_copy(
            tblt_hbm.at[:, pl.ds(g * _LANES, _LANES)], tin.at[b], rsems[b])

    def transpose(b):
        ti = tin.at[b]
        to = tout.at[b]
        rowvs = [(16 * h) % _DIM + iota for h in range(8)]

        @plsc.parallel_loop(0, _DIM, unroll=8)
        def _(u):
            for h in range(8):
                colv = jnp.broadcast_to(2 * u + (h // 4), (16,)).astype(jnp.int32)
                val = plsc.load_gather(ti, [rowvs[h], colv])
                to[u, pl.ds(16 * h, 16)] = val

    def start_write(g, b):
        return pltpu.async_copy(
            tout.at[b], tbl2_hbm.at[pl.ds(g * _DIM, _DIM), :], wsems[b])

    def wait_write(b):
        pltpu.make_async_copy(
            tout.at[b], tbl2_hbm.at[pl.ds(0, _DIM), :], wsems[b]).wait()

    @pl.loop(0, count // 2)
    def _(s):
        descs = []
        for b in range(2):
            i = 2 * s + b

            @pl.when(s > 0)
            def _():
                wait_write(b)

            descs.append(start_read(start + i, b))
        for b in range(2):
            i = 2 * s + b
            descs[b].wait()
            transpose(b)
            start_write(start + i, b)

    for b in range(2):
        wait_write(b)


def kernel(input, table):
    B, L = input.shape
    V, D = table.shape
    n = B * L
    idx = input.reshape(n).astype(jnp.int32)

    full_cols = V // _LANES
    tail_start = full_cols * _LANES
    tail_rows2 = (V - tail_start) // 2

    table_t = jnp.swapaxes(table, 0, 1)
    tail2 = lax.slice(table, (tail_start, 0), (V, D)).reshape(tail_rows2,
                                                             2 * _DIM)
    mesh = plsc.VectorSubcoreMesh(core_axis_name="c", subcore_axis_name="s")
    relayout = pl.kernel(
        _relayout_body,
        out_type=jax.ShapeDtypeStruct((V // 2, 2 * _DIM), jnp.float32),
        mesh=mesh,
        scratch_types=[
            pltpu.VMEM((2, _DIM, _LANES), jnp.float32),
            pltpu.VMEM((2, _DIM, _LANES), jnp.float32),
            pltpu.VMEM((tail_rows2, 2 * _DIM), jnp.float32),
            pltpu.SemaphoreType.DMA,
            pltpu.SemaphoreType.DMA,
            pltpu.SemaphoreType.DMA,
            pltpu.SemaphoreType.DMA,
        ],
        compiler_params=pltpu.CompilerParams(needs_layout_passes=False),
    )
    tbl2 = relayout(table_t, tail2)
    gather = pl.kernel(
        _gather_body,
        out_type=jax.ShapeDtypeStruct((L, _DIM, B), jnp.float32),
        mesh=mesh,
        scratch_types=[
            pltpu.VMEM((n // _NW,), jnp.int32),
            pltpu.VMEM((2, _LANES), jnp.int32),
            pltpu.VMEM((2, _LANES), jnp.int32),
            pltpu.VMEM((2, _LANES, _LANES), jnp.float32),
            pltpu.VMEM((2, _DIM, _LANES), jnp.float32),
            pltpu.SemaphoreType.DMA,
            pltpu.SemaphoreType.DMA,
            pltpu.SemaphoreType.DMA,
            pltpu.SemaphoreType.DMA,
        ],
        compiler_params=pltpu.CompilerParams(needs_layout_passes=False),
    )
    out_t = gather(tbl2, idx)
    return out_t.transpose(2, 0, 1)


# speedup vs baseline: 1.1898x; 1.1898x over previous
import jax
import jax.numpy as jnp
from jax import lax
from jax.experimental import pallas as pl
from jax.experimental.pallas import tpu as pltpu
from jax.experimental.pallas import tpu_sc as plsc

_DIM = 64
_LANES = 128
_NC = 2
_NS = 16
_NW = _NC * _NS
_NBUF = 4


def _relayout_body(tblt_hbm, tail_hbm, tbl2_hbm, tin, tout, tailv, *sems):
    D, V = tblt_hbm.shape
    full_cols = V // _LANES
    quads = full_cols // _NBUF
    base_q = quads // _NW
    remq = quads - base_q * _NW

    rsems = sems[:_NBUF]
    wsems = sems[_NBUF:]

    wid = lax.axis_index("s") * _NC + lax.axis_index("c")
    start = _NBUF * (wid * base_q + lax.min(wid, remq))
    count = _NBUF * (base_q + jnp.where(wid < remq, 1, 0))

    iota = lax.iota(jnp.int32, 16)

    @pl.when(wid == _NW - 1)
    def _():
        pltpu.sync_copy(tail_hbm, tailv)
        pltpu.sync_copy(
            tailv, tbl2_hbm.at[pl.ds(full_cols * (_LANES // 2),
                                     tail_hbm.shape[0]), :])

    def start_read(i, b):
        pltpu.async_copy(
            tblt_hbm.at[:, pl.ds((start + i) * _LANES, _LANES)],
            tin.at[b], rsems[b])

    def wait_read(b):
        pltpu.make_async_copy(
            tblt_hbm.at[:, pl.ds(0, _LANES)], tin.at[b], rsems[b]).wait()

    def transpose(b):
        ti = tin.at[b]
        to = tout.at[b]
        rowvs = [(16 * h) % _DIM + iota for h in range(8)]

        @plsc.parallel_loop(0, _DIM, unroll=8)
        def _(u):
            for h in range(8):
                colv = jnp.broadcast_to(2 * u + (h // 4), (16,)).astype(
                    jnp.int32)
                val = plsc.load_gather(ti, [rowvs[h], colv])
                to[u, pl.ds(16 * h, 16)] = val

    def start_write(i, b):
        pltpu.async_copy(
            tout.at[b], tbl2_hbm.at[pl.ds((start + i) * _DIM, _DIM), :],
            wsems[b])

    def wait_write(b):
        pltpu.make_async_copy(
            tout.at[b], tbl2_hbm.at[pl.ds(0, _DIM), :], wsems[b]).wait()

    for t in range(2):
        start_read(t, t)

    @pl.loop(0, count // _NBUF)
    def _(q):
        for j in range(_NBUF):
            t = _NBUF * q + j
            wait_read(j)

            @pl.when(t >= _NBUF)
            def _():
                wait_write(j)

            transpose(j)
            start_write(t, j)

            @pl.when(t + 2 < count)
            def _():
                start_read(t + 2, (j + 2) % _NBUF)

    for j in range(_NBUF):
        wait_write(j)


def _gather_body(tbl_hbm, idx_hbm, out_hbm, idx_slab, glist, par, gbuf, obuf,
                 *sems):
    L = out_hbm.shape[0]
    BT = out_hbm.shape[2]
    bcols = BT // _NW
    mt_per_w = bcols // _LANES
    n_steps = L * mt_per_w
    slab = bcols * L

    gsems = sems[:_NBUF]
    wsems = sems[_NBUF:]

    wid = lax.axis_index("s") * _NC + lax.axis_index("c")
    b0 = wid * bcols

    pltpu.sync_copy(idx_hbm.at[pl.ds(wid * slab, slab)], idx_slab)

    iota = lax.iota(jnp.int32, 16)

    def build_lists(t, b):
        l = t >> 2
        mm = t & (mt_per_w - 1)
        for g in range(8):
            offs = (mm * _LANES + g * 16 + iota) * L + l
            v = plsc.load_gather(idx_slab, [offs])
            row = lax.shift_right_logical(v, 1)
            parv = lax.shift_left(lax.bitwise_and(v, 1), 6)
            glist.at[b][pl.ds(g * 16, 16)] = row
            par.at[b][pl.ds(g * 16, 16)] = parv

    def start_gather(b):
        pltpu.async_copy(tbl_hbm.at[glist.at[b]], gbuf.at[b], gsems[b])

    def wait_gather(b):
        pltpu.make_async_copy(
            tbl_hbm.at[glist.at[b]], gbuf.at[b], gsems[b]).wait()

    def transpose(b):
        par_ref = par.at[b]
        gb = gbuf.at[b]
        ob = obuf.at[b]
        rowvs = [g * 16 + iota for g in range(8)]
        parvs = [plsc.load_gather(par_ref, [rowvs[g]]) for g in range(8)]

        @plsc.parallel_loop(0, _DIM, unroll=8)
        def _(c):
            for g in range(8):
                val = plsc.load_gather(gb, [rowvs[g], parvs[g] + c])
                ob[c, pl.ds(g * 16, 16)] = val

    def start_write(t, b):
        l = t >> 2
        mm = t & (mt_per_w - 1)
        pltpu.async_copy(
            obuf.at[b],
            out_hbm.at[l, :, pl.ds(b0 + mm * _LANES, _LANES)],
            wsems[b],
        )

    def wait_write(b):
        pltpu.make_async_copy(
            obuf.at[b], out_hbm.at[0, :, pl.ds(b0, _LANES)], wsems[b]).wait()

    for t in range(2):
        build_lists(t, t)
        start_gather(t)

    @pl.loop(0, n_steps // _NBUF)
    def _(q):
        for j in range(_NBUF):
            t = _NBUF * q + j
            wait_gather(j)

            @pl.when(t >= _NBUF)
            def _():
                wait_write(j)

            transpose(j)
            start_write(t, j)

            @pl.when(t + 2 < n_steps)
            def _():
                bn = (j + 2) % _NBUF
                build_lists(t + 2, bn)
                start_gather(bn)

    for j in range(_NBUF):
        wait_write(j)


def kernel(input, table):
    B, L = input.shape
    V, D = table.shape
    n = B * L
    idx = input.reshape(n).astype(jnp.int32)

    full_cols = V // _LANES
    tail_start = full_cols * _LANES
    tail_rows2 = (V - tail_start) // 2

    table_t = jnp.swapaxes(table, 0, 1)
    tail2 = lax.slice(table, (tail_start, 0), (V, D)).reshape(tail_rows2,
                                                             2 * _DIM)
    mesh = plsc.VectorSubcoreMesh(core_axis_name="c", subcore_axis_name="s")
    relayout = pl.kernel(
        _relayout_body,
        out_type=jax.ShapeDtypeStruct((V // 2, 2 * _DIM), jnp.float32),
        mesh=mesh,
        scratch_types=[
            pltpu.VMEM((_NBUF, _DIM, _LANES), jnp.float32),
            pltpu.VMEM((_NBUF, _DIM, _LANES), jnp.float32),
            pltpu.VMEM((tail_rows2, 2 * _DIM), jnp.float32),
        ]
        + [pltpu.SemaphoreType.DMA] * (2 * _NBUF),
        compiler_params=pltpu.CompilerParams(needs_layout_passes=False),
    )
    tbl2 = relayout(table_t, tail2)

    gather = pl.kernel(
        _gather_body,
        out_type=jax.ShapeDtypeStruct((L, _DIM, B), jnp.float32),
        mesh=mesh,
        scratch_types=[
            pltpu.VMEM((n // _NW,), jnp.int32),
            pltpu.VMEM((_NBUF, _LANES), jnp.int32),
            pltpu.VMEM((_NBUF, _LANES), jnp.int32),
            pltpu.VMEM((_NBUF, _LANES, _LANES), jnp.float32),
            pltpu.VMEM((_NBUF, _DIM, _LANES), jnp.float32),
        ]
        + [pltpu.SemaphoreType.DMA] * (2 * _NBUF),
        compiler_params=pltpu.CompilerParams(needs_layout_passes=False),
    )
    out_t = gather(tbl2, idx)
    return out_t.transpose(2, 0, 1)
